# baseline (speedup 1.0000x reference)
import functools

import jax
import jax.numpy as jnp
from jax import lax
from jax.experimental import pallas as pl
from jax.experimental.pallas import tpu as pltpu
from jax.experimental.pallas import tpu_sc as plsc

_B, _T, _D, _E, _TOP_K = 4, 4096, 4096, 64, 8
_TM = 1024
_NC, _NS, _L = 2, 16, 16
_NW = _NC * _NS


def _mm_kernel(x_ref, w_ref, out_ref):
    out_ref[...] = jax.lax.dot_general(
        w_ref[...], x_ref[...],
        dimension_numbers=(((1,), (1,)), ((), ())),
        preferred_element_type=jnp.float32,
    )


def _gate_logits_t(xf, W):
    m = xf.shape[0]
    return pl.pallas_call(
        _mm_kernel,
        grid=(m // _TM,),
        in_specs=[
            pl.BlockSpec((_TM, _D), lambda i: (i, 0)),
            pl.BlockSpec((_E, _D), lambda i: (0, 0)),
        ],
        out_specs=pl.BlockSpec((_E, _TM), lambda i: (0, i)),
        out_shape=jax.ShapeDtypeStruct((_E, m), jnp.float32),
        compiler_params=pltpu.CompilerParams(
            dimension_semantics=("arbitrary",),
        ),
    )(xf, W)


def _make_topk_sc(m):
    cols_per_w = m // _NW
    n_groups = cols_per_w // _L
    mesh = plsc.VectorSubcoreMesh(
        core_axis_name="c", subcore_axis_name="s",
        num_cores=_NC, num_subcores=_NS,
    )

    slab = 128
    n_slabs = cols_per_w // slab
    g_per_slab = slab // _L
    unroll = 8

    @functools.partial(
        pl.kernel,
        out_type=[
            jax.ShapeDtypeStruct((_B, _T, _TOP_K), jnp.int32),
            jax.ShapeDtypeStruct((_B, _T, _TOP_K), jnp.float32),
        ],
        mesh=mesh,
        scratch_types=[
            pltpu.VMEM((_E, cols_per_w), jnp.float32),
            pltpu.VMEM((slab, _TOP_K), jnp.int32),
            pltpu.VMEM((slab, _TOP_K), jnp.float32),
            pltpu.VMEM((slab, _TOP_K), jnp.int32),
            pltpu.VMEM((slab, _TOP_K), jnp.float32),
            pltpu.SemaphoreType.DMA,
            pltpu.SemaphoreType.DMA,
        ],
        compiler_params=pltpu.CompilerParams(needs_layout_passes=False),
    )
    def topk_kernel(lg_hbm, oi_hbm, ow_hbm, buf,
                    oi_a, ow_a, oi_b, ow_b, sem_a, sem_b):
        wid = lax.axis_index("s") * _NC + lax.axis_index("c")
        base = wid * cols_per_w
        b = base // _T
        t0 = base % _T
        pltpu.sync_copy(lg_hbm.at[:, pl.ds(base, cols_per_w)], buf)
        iota = lax.broadcasted_iota(jnp.int32, (_L,), 0)

        stages = [(oi_a, ow_a, sem_a), (oi_b, ow_b, sem_b)]
        pending = [None, None]
        for sb in range(n_slabs):
            oi_v, ow_v, sem = stages[sb % 2]
            if pending[sb % 2] is not None:
                for h in pending[sb % 2]:
                    h.wait()

            def group(gl, gcarry, sb=sb, oi_v=oi_v, ow_v=ow_v):
                g = sb * g_per_slab + gl
                loc = gl * _L + iota

                def ebody(i, carry):
                    vs, ix = carry
                    for d in range(unroll):
                        e = i * unroll + d
                        ev = jnp.broadcast_to(e, (_L,))
                        xv = buf[e, pl.ds(g * _L, _L)]
                        c = [xv > vs[j] for j in range(_TOP_K)]
                        nv, ni = [], []
                        for j in range(_TOP_K):
                            iv = jnp.where(c[j], xv, vs[j])
                            ii = jnp.where(c[j], ev, ix[j])
                            if j:
                                iv = jnp.where(c[j - 1], vs[j - 1], iv)
                                ii = jnp.where(c[j - 1], ix[j - 1], ii)
                            nv.append(iv)
                            ni.append(ii)
                        vs, ix = tuple(nv), tuple(ni)
                    return vs, ix

                neg = jnp.full((_L,), -jnp.inf, jnp.float32)
                zero = jnp.zeros((_L,), jnp.int32)
                vs, ix = lax.fori_loop(
                    0, _E // unroll, ebody,
                    ((neg,) * _TOP_K, (zero,) * _TOP_K))

                exps = [jnp.exp(vs[j] - vs[0]) for j in range(_TOP_K)]
                s = exps[0]
                for j in range(1, _TOP_K):
                    s = s + exps[j]
                r = 1.0 / s
                for j in range(_TOP_K):
                    col = jnp.full((_L,), j, jnp.int32)
                    plsc.store_scatter(oi_v, [loc, col], ix[j])
                    plsc.store_scatter(ow_v, [loc, col], exps[j] * r)
                return gcarry

            lax.fori_loop(0, g_per_slab, group, 0)
            t = t0 + sb * slab
            pending[sb % 2] = (
                pltpu.async_copy(oi_v, oi_hbm.at[b, pl.ds(t, slab)], sem),
                pltpu.async_copy(ow_v, ow_hbm.at[b, pl.ds(t, slab)], sem),
            )
        for p in pending:
            if p is not None:
                for h in p:
                    h.wait()

    return topk_kernel


def kernel(x, W):
    m = _B * _T
    xf = x.reshape(m, _D)
    logits_t = _gate_logits_t(xf, W)
    idx, wts = _make_topk_sc(m)(logits_t)
    return idx, wts


# speedup vs baseline: 1.0111x; 1.0111x over previous
import functools

import jax
import jax.numpy as jnp
from jax import lax
from jax.experimental import pallas as pl
from jax.experimental.pallas import tpu as pltpu
from jax.experimental.pallas import tpu_sc as plsc

_B, _T, _D, _E, _TOP_K = 4, 4096, 4096, 64, 8
_TM = 512
_NC, _NS, _L = 2, 16, 16
_NW = _NC * _NS


def _mm_kernel(x_ref, w_ref, out_ref):
    out_ref[...] = jax.lax.dot_general(
        w_ref[...], x_ref[...],
        dimension_numbers=(((1,), (1,)), ((), ())),
        preferred_element_type=jnp.float32,
    )


def _gate_logits_t(xf, W):
    m = xf.shape[0]
    return pl.pallas_call(
        _mm_kernel,
        grid=(m // _TM,),
        in_specs=[
            pl.BlockSpec((_TM, _D), lambda i: (i, 0)),
            pl.BlockSpec((_E, _D), lambda i: (0, 0)),
        ],
        out_specs=pl.BlockSpec((_E, _TM), lambda i: (0, i)),
        out_shape=jax.ShapeDtypeStruct((_E, m), jnp.float32),
        compiler_params=pltpu.CompilerParams(
            dimension_semantics=("arbitrary",),
        ),
    )(xf, W)


def _make_topk_sc(m):
    cols_per_w = m // _NW
    n_groups = cols_per_w // _L
    mesh = plsc.VectorSubcoreMesh(
        core_axis_name="c", subcore_axis_name="s",
        num_cores=_NC, num_subcores=_NS,
    )

    slab = 128
    n_slabs = cols_per_w // slab
    g_per_slab = slab // _L
    unroll = 4

    @functools.partial(
        pl.kernel,
        out_type=[
            jax.ShapeDtypeStruct((_B, _T, _TOP_K), jnp.int32),
            jax.ShapeDtypeStruct((_B, _T, _TOP_K), jnp.float32),
        ],
        mesh=mesh,
        scratch_types=[
            pltpu.VMEM((_E, cols_per_w), jnp.float32),
            pltpu.VMEM((slab, _TOP_K), jnp.int32),
            pltpu.VMEM((slab, _TOP_K), jnp.float32),
            pltpu.VMEM((slab, _TOP_K), jnp.int32),
            pltpu.VMEM((slab, _TOP_K), jnp.float32),
            pltpu.SemaphoreType.DMA,
            pltpu.SemaphoreType.DMA,
        ],
        compiler_params=pltpu.CompilerParams(needs_layout_passes=False),
    )
    def topk_kernel(lg_hbm, oi_hbm, ow_hbm, buf,
                    oi_a, ow_a, oi_b, ow_b, sem_a, sem_b):
        wid = lax.axis_index("s") * _NC + lax.axis_index("c")
        base = wid * cols_per_w
        b = base // _T
        t0 = base % _T
        pltpu.sync_copy(lg_hbm.at[:, pl.ds(base, cols_per_w)], buf)
        iota = lax.broadcasted_iota(jnp.int32, (_L,), 0)

        stages = [(oi_a, ow_a, sem_a), (oi_b, ow_b, sem_b)]
        pending = [None, None]
        for sb in range(n_slabs):
            oi_v, ow_v, sem = stages[sb % 2]
            if pending[sb % 2] is not None:
                for h in pending[sb % 2]:
                    h.wait()

            def group(gl, gcarry, sb=sb, oi_v=oi_v, ow_v=ow_v):
                g = sb * g_per_slab + gl
                loc = gl * _L + iota

                def ebody(i, carry):
                    vs, ix = carry
                    for d in range(unroll):
                        e = i * unroll + d
                        ev = jnp.broadcast_to(e, (_L,))
                        xv = buf[e, pl.ds(g * _L, _L)]
                        c = [xv > vs[j] for j in range(_TOP_K)]
                        nv, ni = [], []
                        for j in range(_TOP_K):
                            iv = jnp.where(c[j], xv, vs[j])
                            ii = jnp.where(c[j], ev, ix[j])
                            if j:
                                iv = jnp.where(c[j - 1], vs[j - 1], iv)
                                ii = jnp.where(c[j - 1], ix[j - 1], ii)
                            nv.append(iv)
                            ni.append(ii)
                        vs, ix = tuple(nv), tuple(ni)
                    return vs, ix

                neg = jnp.full((_L,), -jnp.inf, jnp.float32)
                zero = jnp.zeros((_L,), jnp.int32)
                vs, ix = lax.fori_loop(
                    0, _E // unroll, ebody,
                    ((neg,) * _TOP_K, (zero,) * _TOP_K))

                exps = [jnp.exp(vs[j] - vs[0]) for j in range(_TOP_K)]
                s = exps[0]
                for j in range(1, _TOP_K):
                    s = s + exps[j]
                r = 1.0 / s
                for j in range(_TOP_K):
                    col = jnp.full((_L,), j, jnp.int32)
                    plsc.store_scatter(oi_v, [loc, col], ix[j])
                    plsc.store_scatter(ow_v, [loc, col], exps[j] * r)
                return gcarry

            lax.fori_loop(0, g_per_slab, group, 0)
            t = t0 + sb * slab
            pending[sb % 2] = (
                pltpu.async_copy(oi_v, oi_hbm.at[b, pl.ds(t, slab)], sem),
                pltpu.async_copy(ow_v, ow_hbm.at[b, pl.ds(t, slab)], sem),
            )
        for p in pending:
            if p is not None:
                for h in p:
                    h.wait()

    return topk_kernel


def kernel(x, W):
    m = _B * _T
    xf = x.reshape(m, _D)
    logits_t = _gate_logits_t(xf, W)
    idx, wts = _make_topk_sc(m)(logits_t)
    return idx, wts


# speedup vs baseline: 1.0121x; 1.0010x over previous
import functools

import jax
import jax.numpy as jnp
from jax import lax
from jax.experimental import pallas as pl
from jax.experimental.pallas import tpu as pltpu
from jax.experimental.pallas import tpu_sc as plsc

_B, _T, _D, _E, _TOP_K = 4, 4096, 4096, 64, 8
_TM = 512
_NC, _NS, _L = 2, 16, 16
_NW = _NC * _NS


def _mm_kernel(x_ref, w_ref, out_ref):
    out_ref[...] = jax.lax.dot_general(
        w_ref[...], x_ref[...],
        dimension_numbers=(((1,), (1,)), ((), ())),
        preferred_element_type=jnp.float32,
    )


def _gate_logits_t(xf, W):
    m = xf.shape[0]
    return pl.pallas_call(
        _mm_kernel,
        grid=(m // _TM,),
        in_specs=[
            pl.BlockSpec((_TM, _D), lambda i: (i, 0)),
            pl.BlockSpec((_E, _D), lambda i: (0, 0)),
        ],
        out_specs=pl.BlockSpec((_E, _TM), lambda i: (0, i)),
        out_shape=jax.ShapeDtypeStruct((_E, m), jnp.float32),
        compiler_params=pltpu.CompilerParams(
            dimension_semantics=("arbitrary",),
        ),
    )(xf, W)


def _make_topk_sc(m):
    cols_per_w = m // _NW
    n_groups = cols_per_w // _L
    mesh = plsc.VectorSubcoreMesh(
        core_axis_name="c", subcore_axis_name="s",
        num_cores=_NC, num_subcores=_NS,
    )

    slab = 128
    n_slabs = cols_per_w // slab
    g_per_slab = slab // _L
    unroll = 4

    @functools.partial(
        pl.kernel,
        out_type=[
            jax.ShapeDtypeStruct((_B, _T, _TOP_K), jnp.int32),
            jax.ShapeDtypeStruct((_B, _T, _TOP_K), jnp.float32),
        ],
        mesh=mesh,
        scratch_types=[
            pltpu.VMEM((_E, cols_per_w), jnp.float32),
            pltpu.VMEM((slab, _TOP_K), jnp.int32),
            pltpu.VMEM((slab, _TOP_K), jnp.float32),
            pltpu.VMEM((slab, _TOP_K), jnp.int32),
            pltpu.VMEM((slab, _TOP_K), jnp.float32),
            pltpu.SemaphoreType.DMA,
            pltpu.SemaphoreType.DMA,
            pltpu.SemaphoreType.DMA,
        ],
        compiler_params=pltpu.CompilerParams(needs_layout_passes=False),
    )
    def topk_kernel(lg_hbm, oi_hbm, ow_hbm, buf,
                    oi_a, ow_a, oi_b, ow_b, sem_a, sem_b, sem_in):
        wid = lax.axis_index("s") * _NC + lax.axis_index("c")
        base = wid * cols_per_w
        b = base // _T
        t0 = base % _T
        half = cols_per_w // 2
        in0 = pltpu.async_copy(
            lg_hbm.at[:, pl.ds(base, half)], buf.at[:, pl.ds(0, half)],
            sem_in)
        in1 = pltpu.async_copy(
            lg_hbm.at[:, pl.ds(base + half, half)],
            buf.at[:, pl.ds(half, half)], sem_in)
        in0.wait()
        iota = lax.broadcasted_iota(jnp.int32, (_L,), 0)

        stages = [(oi_a, ow_a, sem_a), (oi_b, ow_b, sem_b)]
        pending = [None, None]
        for sb in range(n_slabs):
            if sb == n_slabs // 2:
                in1.wait()
            oi_v, ow_v, sem = stages[sb % 2]
            if pending[sb % 2] is not None:
                for h in pending[sb % 2]:
                    h.wait()

            def group(gl, gcarry, sb=sb, oi_v=oi_v, ow_v=ow_v):
                g = sb * g_per_slab + gl
                loc = gl * _L + iota

                def ebody(i, carry):
                    vs, ix = carry
                    for d in range(unroll):
                        e = i * unroll + d
                        ev = jnp.broadcast_to(e, (_L,))
                        xv = buf[e, pl.ds(g * _L, _L)]
                        c = [xv > vs[j] for j in range(_TOP_K)]
                        nv, ni = [], []
                        for j in range(_TOP_K):
                            iv = jnp.where(c[j], xv, vs[j])
                            ii = jnp.where(c[j], ev, ix[j])
                            if j:
                                iv = jnp.where(c[j - 1], vs[j - 1], iv)
                                ii = jnp.where(c[j - 1], ix[j - 1], ii)
                            nv.append(iv)
                            ni.append(ii)
                        vs, ix = tuple(nv), tuple(ni)
                    return vs, ix

                neg = jnp.full((_L,), -jnp.inf, jnp.float32)
                zero = jnp.zeros((_L,), jnp.int32)
                vs, ix = lax.fori_loop(
                    0, _E // unroll, ebody,
                    ((neg,) * _TOP_K, (zero,) * _TOP_K))

                exps = [jnp.exp(vs[j] - vs[0]) for j in range(_TOP_K)]
                s = exps[0]
                for j in range(1, _TOP_K):
                    s = s + exps[j]
                r = 1.0 / s
                for j in range(_TOP_K):
                    col = jnp.full((_L,), j, jnp.int32)
                    plsc.store_scatter(oi_v, [loc, col], ix[j])
                    plsc.store_scatter(ow_v, [loc, col], exps[j] * r)
                return gcarry

            lax.fori_loop(0, g_per_slab, group, 0)
            t = t0 + sb * slab
            pending[sb % 2] = (
                pltpu.async_copy(oi_v, oi_hbm.at[b, pl.ds(t, slab)], sem),
                pltpu.async_copy(ow_v, ow_hbm.at[b, pl.ds(t, slab)], sem),
            )
        for p in pending:
            if p is not None:
                for h in p:
                    h.wait()

    return topk_kernel


def kernel(x, W):
    m = _B * _T
    xf = x.reshape(m, _D)
    logits_t = _gate_logits_t(xf, W)
    idx, wts = _make_topk_sc(m)(logits_t)
    return idx, wts
